# baseline (speedup 1.0000x reference)
import functools

import jax
import jax.numpy as jnp
from jax import lax
from jax.experimental import pallas as pl
from jax.experimental.pallas import tpu as pltpu
from jax.experimental.pallas import tpu_sc as plsc

N = 50000
E = 1600000
D = 31
L = 5
DP = 32
HF = 16

NS = 16
BATCH = 128
ROWS_PER_TILE = 800
E_PAD = NS * ROWS_PER_TILE * BATCH
CHUNK = 200
N_CHUNKS = ROWS_PER_TILE // CHUNK
NB = 8
N_TILE = 3128
N_PAD = NS * N_TILE


def _sc_aggregate_body(h_hbm, idx_hbm, z_hbm, out_hbm, src_v, dst_v, agg_s,
                       *bs):
    bufs, sems = bs[:NB], bs[NB:]
    c = lax.axis_index("c")
    s = lax.axis_index("s")

    pltpu.sync_copy(z_hbm, agg_s.at[pl.ds(s * N_TILE, N_TILE)])
    plsc.subcore_barrier()

    def fire(i, r):
        pltpu.async_copy(h_hbm.at[src_v.at[r]], bufs[i], sems[i])

    def drain(i, r):
        pltpu.make_async_copy(h_hbm.at[src_v.at[r]], bufs[i], sems[i]).wait()
        pltpu.sync_copy(bufs[i], agg_s.at[dst_v.at[r]], add=True)

    for k in range(N_CHUNKS):
        pltpu.sync_copy(idx_hbm.at[2 * c, s, pl.ds(k * CHUNK, CHUNK)], src_v)
        pltpu.sync_copy(idx_hbm.at[1, s, pl.ds(k * CHUNK, CHUNK)], dst_v)

        for i in range(NB):
            fire(i, i)

        def step(it, carry):
            base = NB * it
            for i in range(NB):
                drain(i, base + i)
                fire(i, base + NB + i)
            return carry

        lax.fori_loop(0, CHUNK // NB - 1, step, 0)
        for i in range(NB):
            drain(i, CHUNK - NB + i)

    plsc.subcore_barrier()
    pltpu.sync_copy(agg_s.at[pl.ds(s * N_TILE, N_TILE)],
                    out_hbm.at[c, pl.ds(s * N_TILE, N_TILE)])


_sc_aggregate = functools.partial(
    pl.kernel,
    out_type=jax.ShapeDtypeStruct((2, N_PAD, HF), jnp.float32),
    mesh=plsc.VectorSubcoreMesh(core_axis_name="c", subcore_axis_name="s"),
    scratch_types=[
        pltpu.VMEM((CHUNK, BATCH), jnp.int32),
        pltpu.VMEM((CHUNK, BATCH), jnp.int32),
        pltpu.VMEM_SHARED((N_PAD, HF), jnp.float32),
    ] + [pltpu.VMEM((BATCH, HF), jnp.float32) for _ in range(NB)]
      + [pltpu.SemaphoreType.DMA for _ in range(NB)],
    compiler_params=pltpu.CompilerParams(use_tc_tiling_on_sc=False),
)(_sc_aggregate_body)


def _tc_dense_body(h_ref, a_ref, ws_ref, wn_ref, b_ref, out_ref):
    a = a_ref[...]
    deg = a[D:DP, :]
    inv = 1.0 / jnp.maximum(deg, 1.0)
    dot = functools.partial(jnp.dot, preferred_element_type=jnp.float32)
    pre = (dot(ws_ref[...], h_ref[...]) + dot(wn_ref[...], a * inv)
           + b_ref[...])
    out_ref[...] = jnp.maximum(pre, 0.0)


_BLK = 2176


def _tc_dense(h_t, agg_t, ws_t, wn_t, b_col):
    return pl.pallas_call(
        _tc_dense_body,
        grid=(N_PAD // _BLK,),
        in_specs=[
            pl.BlockSpec((DP, _BLK), lambda i: (0, i)),
            pl.BlockSpec((DP, _BLK), lambda i: (0, i)),
            pl.BlockSpec((DP, DP), lambda i: (0, 0)),
            pl.BlockSpec((DP, DP), lambda i: (0, 0)),
            pl.BlockSpec((DP, 1), lambda i: (0, 0)),
        ],
        out_specs=pl.BlockSpec((DP, _BLK), lambda i: (0, i)),
        out_shape=jax.ShapeDtypeStruct((DP, N_PAD), jnp.float32),
    )(h_t, agg_t, ws_t, wn_t, b_col)


def kernel(x, edge_index, W_self, W_neigh, b):
    src = edge_index[0].astype(jnp.int32)
    dst = edge_index[1].astype(jnp.int32)

    pad = E_PAD - E
    fill = jnp.arange(pad, dtype=jnp.int32)
    src_p = jnp.concatenate([src, (fill * 97) % N])
    dst_p = jnp.concatenate([dst, N + fill % (N_PAD - N)])
    idx = jnp.stack([src_p, dst_p, src_p + N_PAD]).reshape(
        3, NS, ROWS_PER_TILE, BATCH)

    xp = jnp.pad(x, ((0, N_PAD - N), (0, 0)))
    h_t = jnp.concatenate([xp, jnp.ones((N_PAD, 1), jnp.float32)], axis=1).T

    ws_t = jnp.swapaxes(jnp.pad(W_self, ((0, 0), (0, 1), (0, 1))), 1, 2)
    wn_t = jnp.swapaxes(jnp.pad(W_neigh, ((0, 0), (0, 1), (0, 1))), 1, 2)
    b_p = jnp.pad(b, ((0, 0), (0, 1))).at[:, D].set(1.0)

    z = jnp.zeros((N_TILE, HF), jnp.float32)

    def layer(h_tc, wsb):
        ws_i, wn_i, b_i = wsb
        h_flat = jnp.concatenate([h_tc[:HF].T, h_tc[HF:].T], axis=0)
        agg = _sc_aggregate(h_flat, idx, z)
        agg_t = jnp.concatenate([agg[0], agg[1]], axis=1).T
        return _tc_dense(h_tc, agg_t, ws_i, wn_i, b_i), None

    h_t, _ = lax.scan(layer, h_t, (ws_t, wn_t, b_p[:, :, None]))
    return h_t[:D, :N].T
